# baseline (speedup 1.0000x reference)
import functools

import jax
import jax.numpy as jnp
from jax import lax
from jax.experimental import pallas as pl
from jax.experimental.pallas import tpu as pltpu
from jax.experimental.pallas import tpu_sc as plsc

_K = 1024
_D = 256
_TOK_BLK = 4608

_NW = 32
_NTOK = 4608
_BPW = _NTOK // _NW
_UCHUNKS = _NTOK // 16 // 16


def _dist_argmin_body(xsq_ref, esq_ref, x_ref, emb_ref, idx_ref):
    x = x_ref[...]
    t = x.shape[0]
    cross = lax.dot_general(
        x, emb_ref[...], (((1,), (1,)), ((), ())),
        preferred_element_type=jnp.float32)
    xsq_col = xsq_ref[...].reshape(t, 1)
    esq_row = esq_ref[...].reshape(1, _K)
    s = (xsq_col + esq_row) - 2.0 * cross
    dist = jnp.sqrt(jnp.maximum(s, 0.0))
    m = jnp.min(dist, axis=-1, keepdims=True)
    iota = lax.broadcasted_iota(jnp.int32, (t, _K), 1).astype(jnp.float32)
    idx_f = jnp.min(jnp.where(dist == m, iota, float(_K)), axis=-1,
                    keepdims=True)
    idx_ref[...] = idx_f.astype(jnp.int32)


def _dist_argmin(xsq3, esq2, x2, embedding, interpret=False):
    grid = _NTOK // _TOK_BLK
    return pl.pallas_call(
        _dist_argmin_body,
        grid=(grid,),
        in_specs=[
            pl.BlockSpec((1, 1, _TOK_BLK), lambda i: (i, 0, 0)),
            pl.BlockSpec((1, _K), lambda i: (0, 0)),
            pl.BlockSpec((_TOK_BLK, _D), lambda i: (i, 0)),
            pl.BlockSpec((_K, _D), lambda i: (0, 0)),
        ],
        out_specs=pl.BlockSpec((_TOK_BLK, 1), lambda i: (i, 0)),
        out_shape=jax.ShapeDtypeStruct((_NTOK, 1), jnp.int32),
        interpret=interpret,
    )(xsq3, esq2, x2, embedding)


def _sc_post(embedding, idx_flat):
    mesh = plsc.VectorSubcoreMesh(core_axis_name="c", subcore_axis_name="s")

    @functools.partial(
        pl.kernel,
        mesh=mesh,
        compiler_params=pltpu.CompilerParams(needs_layout_passes=False),
        out_type=[
            jax.ShapeDtypeStruct((_NTOK, _D), jnp.float32),
            jax.ShapeDtypeStruct((16,), jnp.float32),
        ],
        scratch_types=[
            pltpu.VMEM((_BPW,), jnp.int32),
            pltpu.VMEM((_BPW, _D), jnp.float32),
            pltpu.VMEM((_NTOK,), jnp.int32),
            pltpu.VMEM((_K,), jnp.float32),
            pltpu.VMEM((16,), jnp.float32),
            pltpu.SemaphoreType.DMA,
        ],
    )
    def k(emb_hbm, idx_hbm, out_hbm, usage_hbm, idx_v, rows_v,
          idxu_v, occ_v, usage_v, sem):
        cid = lax.axis_index("c")
        sid = lax.axis_index("s")
        wid = sid * 2 + cid
        base = wid * _BPW
        pltpu.sync_copy(idx_hbm.at[pl.ds(base, _BPW)], idx_v)
        gat = pltpu.async_copy(emb_hbm.at[idx_v], rows_v, sem)

        zeros16 = jnp.zeros((16,), jnp.float32)
        ones16 = jnp.ones((16,), jnp.float32)

        @pl.when((cid == 0) & (sid == 0))
        def _usage():
            pltpu.sync_copy(idx_hbm, idxu_v)

            def _zrow(i, c):
                for j in range(8):
                    occ_v[pl.ds(i * 128 + j * 16, 16)] = zeros16
                return c

            lax.fori_loop(0, _K // 128, _zrow, 0)

            def _srow(i, c):
                for j in range(8):
                    iv = idxu_v[pl.ds(i * 128 + j * 16, 16)]
                    plsc.store_scatter(occ_v, [iv], ones16)
                return c

            lax.fori_loop(0, _NTOK // 128, _srow, 0)

            def _crow(i, acc):
                for j in range(8):
                    v = occ_v[pl.ds(i * 128 + j * 16, 16)]
                    acc = acc + jnp.where(v == 0.0, 1.0, 0.0)
                return acc

            acc = lax.fori_loop(0, _K // 128, _crow, zeros16)
            tot = jnp.sum(acc)
            usage_v[...] = zeros16 + tot * (1.0 / _K)
            pltpu.sync_copy(usage_v, usage_hbm)

        gat.wait()
        pltpu.sync_copy(rows_v, out_hbm.at[pl.ds(base, _BPW)])

    return k(embedding, idx_flat)


def kernel(x, embedding):
    x = x.astype(jnp.float32)
    b, n, _ = x.shape
    x_sq = jnp.sum(x * x, axis=-1)
    e_sq = jnp.sum(embedding * embedding, axis=-1)
    x2 = x.reshape(_NTOK, _D)
    xsq3 = x_sq.reshape(_NTOK // _TOK_BLK, 1, _TOK_BLK)
    idx_flat = _dist_argmin(xsq3, e_sq.reshape(1, _K), x2,
                            embedding).reshape(_NTOK)
    quantized, usage16 = _sc_post(embedding, idx_flat)
    return (quantized.reshape(b, n, _D), idx_flat.reshape(b, n),
            usage16[0].reshape(()))


# speedup vs baseline: 1.0080x; 1.0080x over previous
import functools

import jax
import jax.numpy as jnp
from jax import lax
from jax.experimental import pallas as pl
from jax.experimental.pallas import tpu as pltpu
from jax.experimental.pallas import tpu_sc as plsc

_K = 1024
_D = 256
_TOK_BLK = 2304

_NW = 32
_NTOK = 4608
_BPW = _NTOK // _NW
_UCHUNKS = _NTOK // 16 // 16


def _dist_argmin_body(xsq_ref, esq_ref, x_ref, emb_ref, idx_ref):
    x = x_ref[...]
    t = x.shape[0]
    cross = lax.dot_general(
        x, emb_ref[...], (((1,), (1,)), ((), ())),
        preferred_element_type=jnp.float32)
    xsq_col = xsq_ref[...].reshape(t, 1)
    esq_row = esq_ref[...].reshape(1, _K)
    s = (xsq_col + esq_row) - 2.0 * cross
    dist = jnp.sqrt(jnp.maximum(s, 0.0))
    m = jnp.min(dist, axis=-1, keepdims=True)
    iota = lax.broadcasted_iota(jnp.int32, (t, _K), 1).astype(jnp.float32)
    idx_f = jnp.min(jnp.where(dist == m, iota, float(_K)), axis=-1,
                    keepdims=True)
    idx_ref[...] = idx_f.astype(jnp.int32)


def _dist_argmin(xsq3, esq2, x2, embedding, interpret=False):
    grid = _NTOK // _TOK_BLK
    return pl.pallas_call(
        _dist_argmin_body,
        grid=(grid,),
        in_specs=[
            pl.BlockSpec((1, 1, _TOK_BLK), lambda i: (i, 0, 0)),
            pl.BlockSpec((1, _K), lambda i: (0, 0)),
            pl.BlockSpec((_TOK_BLK, _D), lambda i: (i, 0)),
            pl.BlockSpec((_K, _D), lambda i: (0, 0)),
        ],
        out_specs=pl.BlockSpec((_TOK_BLK, 1), lambda i: (i, 0)),
        out_shape=jax.ShapeDtypeStruct((_NTOK, 1), jnp.int32),
        interpret=interpret,
    )(xsq3, esq2, x2, embedding)


def _sc_post(embedding, idx_flat):
    mesh = plsc.VectorSubcoreMesh(core_axis_name="c", subcore_axis_name="s")

    @functools.partial(
        pl.kernel,
        mesh=mesh,
        compiler_params=pltpu.CompilerParams(needs_layout_passes=False),
        out_type=[
            jax.ShapeDtypeStruct((_NTOK, _D), jnp.float32),
            jax.ShapeDtypeStruct((16,), jnp.float32),
        ],
        scratch_types=[
            pltpu.VMEM((_BPW,), jnp.int32),
            pltpu.VMEM((_BPW, _D), jnp.float32),
            pltpu.VMEM((_NTOK,), jnp.int32),
            pltpu.VMEM((_K,), jnp.float32),
            pltpu.VMEM((16,), jnp.float32),
            pltpu.SemaphoreType.DMA,
        ],
    )
    def k(emb_hbm, idx_hbm, out_hbm, usage_hbm, idx_v, rows_v,
          idxu_v, occ_v, usage_v, sem):
        cid = lax.axis_index("c")
        sid = lax.axis_index("s")
        wid = sid * 2 + cid
        base = wid * _BPW
        pltpu.sync_copy(idx_hbm.at[pl.ds(base, _BPW)], idx_v)
        gat = pltpu.async_copy(emb_hbm.at[idx_v], rows_v, sem)

        zeros16 = jnp.zeros((16,), jnp.float32)
        ones16 = jnp.ones((16,), jnp.float32)

        @pl.when((cid == 0) & (sid == 0))
        def _usage():
            pltpu.sync_copy(idx_hbm, idxu_v)

            def _zrow(i, c):
                for j in range(8):
                    occ_v[pl.ds(i * 128 + j * 16, 16)] = zeros16
                return c

            lax.fori_loop(0, _K // 128, _zrow, 0)

            def _srow(i, c):
                for j in range(8):
                    iv = idxu_v[pl.ds(i * 128 + j * 16, 16)]
                    plsc.store_scatter(occ_v, [iv], ones16)
                return c

            lax.fori_loop(0, _NTOK // 128, _srow, 0)

            def _crow(i, acc):
                for j in range(8):
                    v = occ_v[pl.ds(i * 128 + j * 16, 16)]
                    acc = acc + jnp.where(v == 0.0, 1.0, 0.0)
                return acc

            acc = lax.fori_loop(0, _K // 128, _crow, zeros16)
            tot = jnp.sum(acc)
            usage_v[...] = zeros16 + tot * (1.0 / _K)
            pltpu.sync_copy(usage_v, usage_hbm)

        gat.wait()
        pltpu.sync_copy(rows_v, out_hbm.at[pl.ds(base, _BPW)])

    return k(embedding, idx_flat)


def kernel(x, embedding):
    x = x.astype(jnp.float32)
    b, n, _ = x.shape
    x_sq = jnp.sum(x * x, axis=-1)
    e_sq = jnp.sum(embedding * embedding, axis=-1)
    x2 = x.reshape(_NTOK, _D)
    xsq3 = x_sq.reshape(_NTOK // _TOK_BLK, 1, _TOK_BLK)
    idx_flat = _dist_argmin(xsq3, e_sq.reshape(1, _K), x2,
                            embedding).reshape(_NTOK)
    quantized, usage16 = _sc_post(embedding, idx_flat)
    return (quantized.reshape(b, n, _D), idx_flat.reshape(b, n),
            usage16[0].reshape(()))
